# Initial kernel scaffold; baseline (speedup 1.0000x reference)
import jax
import jax.numpy as jnp
from jax.experimental import pallas as pl


def kernel(session_embedding, batch, item_weight, W1, b1, W2, b2, Wq, bq, W3, b3):
    raise NotImplementedError("write your pallas kernel here")


# speedup vs baseline: 3.5842x; 3.5842x over previous
import jax
import jax.numpy as jnp
from jax import lax
from jax.experimental import pallas as pl
from jax.experimental.pallas import tpu as pltpu

_B = 16
_TB = 2048
_VB = 2048


def _seg_body(bq_ref, batch_ref, se_ref, vn_ref, W1_ref, W2_ref, Wqb_ref,
              bias_ref, W3_ref, b3_ref, sh_ref, acc_ref):
    i = pl.program_id(0)
    nb = pl.num_programs(0)
    se = se_ref[...]
    tb = se.shape[0]
    brow = batch_ref[0]
    seg_iota = lax.broadcasted_iota(jnp.int32, (_B, tb), 0)
    ohT = (jnp.broadcast_to(brow, (_B, tb)) == seg_iota
           ).astype(jnp.float32)
    c = lax.dot_general(vn_ref[...], W1_ref[...], (((1,), (1,)), ((), ())),
                        preferred_element_type=jnp.float32) + bias_ref[...]
    m = lax.dot_general(se, W2_ref[...], (((1,), (1,)), ((), ())),
                        preferred_element_type=jnp.float32)
    g = lax.dot_general(ohT, c, (((0,), (0,)), ((), ())),
                        preferred_element_type=jnp.float32)
    h = jax.nn.sigmoid(m + g)
    alphaT = lax.dot_general(Wqb_ref[...], h, (((1,), (1,)), ((), ())),
                             preferred_element_type=jnp.float32) + bq_ref[0]
    wT = ohT * alphaT
    part = lax.dot_general(wT, se, (((1,), (0,)), ((), ())),
                           preferred_element_type=jnp.float32)

    @pl.when(i == 0)
    def _():
        acc_ref[...] = jnp.zeros_like(acc_ref)

    acc_ref[...] += part

    @pl.when(i == nb - 1)
    def _():
        cat = jnp.concatenate([vn_ref[...], acc_ref[...]], axis=1)
        sh_ref[...] = lax.dot_general(
            cat, W3_ref[...], (((1,), (1,)), ((), ())),
            preferred_element_type=jnp.float32) + b3_ref[...]


def _score_body(sh_ref, iw_ref, z_ref):
    z_ref[...] = lax.dot_general(sh_ref[...], iw_ref[...],
                                 (((1,), (1,)), ((), ())),
                                 preferred_element_type=jnp.float32)


def kernel(session_embedding, batch, item_weight, W1, b1, W2, b2, Wq, bq, W3, b3):
    T, H = session_embedding.shape
    V = item_weight.shape[0]
    batch32 = batch.astype(jnp.int32)

    last = jnp.searchsorted(batch32, jnp.arange(_B, dtype=jnp.int32),
                            side="right").astype(jnp.int32) - 1
    v_n = jnp.take(session_embedding, last, axis=0, mode="clip")

    nb = T // _TB
    batch3 = batch32.reshape(nb, 1, _TB)
    bias = (b1 + b2).reshape(1, H)
    Wqb = jnp.broadcast_to(Wq, (_B, H))

    sh = pl.pallas_call(
        _seg_body,
        grid=(nb,),
        in_specs=[
            pl.BlockSpec(memory_space=pltpu.SMEM),
            pl.BlockSpec((1, 1, _TB), lambda i: (i, 0, 0)),
            pl.BlockSpec((_TB, H), lambda i: (i, 0)),
            pl.BlockSpec((_B, H), lambda i: (0, 0)),
            pl.BlockSpec((H, H), lambda i: (0, 0)),
            pl.BlockSpec((H, H), lambda i: (0, 0)),
            pl.BlockSpec((_B, H), lambda i: (0, 0)),
            pl.BlockSpec((1, H), lambda i: (0, 0)),
            pl.BlockSpec((H, 2 * H), lambda i: (0, 0)),
            pl.BlockSpec((1, H), lambda i: (0, 0)),
        ],
        out_specs=pl.BlockSpec((_B, H), lambda i: (0, 0)),
        out_shape=jax.ShapeDtypeStruct((_B, H), jnp.float32),
        scratch_shapes=[pltpu.VMEM((_B, H), jnp.float32)],
    )(bq, batch3, session_embedding, v_n, W1, W2, Wqb, bias,
      W3, b3.reshape(1, H))

    nv = pl.cdiv(V, _VB)
    z = pl.pallas_call(
        _score_body,
        grid=(nv,),
        in_specs=[
            pl.BlockSpec((_B, H), lambda j: (0, 0)),
            pl.BlockSpec((_VB, H), lambda j: (j, 0)),
        ],
        out_specs=pl.BlockSpec((_B, _VB), lambda j: (0, j)),
        out_shape=jax.ShapeDtypeStruct((_B, V), jnp.float32),
    )(sh, item_weight)
    return z


# speedup vs baseline: 3.7435x; 1.0445x over previous
import jax
import jax.numpy as jnp
from jax import lax
from jax.experimental import pallas as pl
from jax.experimental.pallas import tpu as pltpu

_B = 16
_TB = 2048
_VB = 2048


def _make_body(nb):
    def _body(bq_ref, batch_ref, se_ref, vn_ref, W1_ref, W2_ref, Wqb_ref,
              bias_ref, W3_ref, b3_ref, iw_ref, z_ref, acc_ref, sh_ref):
        i = pl.program_id(0)

        @pl.when(i < nb)
        def _token():
            se = se_ref[...]
            tb = se.shape[0]
            brow = batch_ref[0]
            seg_iota = lax.broadcasted_iota(jnp.int32, (_B, tb), 0)
            ohT = (jnp.broadcast_to(brow, (_B, tb)) == seg_iota
                   ).astype(jnp.bfloat16)
            c = lax.dot_general(vn_ref[...], W1_ref[...],
                                (((1,), (1,)), ((), ())),
                                preferred_element_type=jnp.float32)
            c = (c + bias_ref[...]).astype(jnp.bfloat16)
            m = lax.dot_general(se, W2_ref[...], (((1,), (1,)), ((), ())),
                                preferred_element_type=jnp.float32)
            g = lax.dot_general(ohT, c, (((0,), (0,)), ((), ())),
                                preferred_element_type=jnp.float32)
            h = jax.nn.sigmoid(m + g).astype(jnp.bfloat16)
            alphaT = lax.dot_general(Wqb_ref[...], h, (((1,), (1,)), ((), ())),
                                     preferred_element_type=jnp.float32)
            wT = (ohT.astype(jnp.float32) * (alphaT + bq_ref[0])
                  ).astype(jnp.bfloat16)
            part = lax.dot_general(wT, se.astype(jnp.bfloat16),
                                   (((1,), (0,)), ((), ())),
                                   preferred_element_type=jnp.float32)

            @pl.when(i == 0)
            def _():
                acc_ref[...] = jnp.zeros_like(acc_ref)

            acc_ref[...] += part

            @pl.when(i == nb - 1)
            def _():
                cat = jnp.concatenate([vn_ref[...], acc_ref[...]], axis=1)
                sh_ref[...] = lax.dot_general(
                    cat, W3_ref[...], (((1,), (1,)), ((), ())),
                    preferred_element_type=jnp.float32) + b3_ref[...]

        @pl.when(i >= nb)
        def _vocab():
            z_ref[...] = lax.dot_general(sh_ref[...], iw_ref[...],
                                         (((1,), (1,)), ((), ())),
                                         preferred_element_type=jnp.float32)

    return _body


def kernel(session_embedding, batch, item_weight, W1, b1, W2, b2, Wq, bq, W3, b3):
    T, H = session_embedding.shape
    V = item_weight.shape[0]
    batch32 = batch.astype(jnp.int32)

    last = jnp.searchsorted(batch32, jnp.arange(_B, dtype=jnp.int32),
                            side="right").astype(jnp.int32) - 1
    v_n = jnp.take(session_embedding, last, axis=0, mode="clip")

    nb = T // _TB
    nv = pl.cdiv(V, _VB)
    batch3 = batch32.reshape(nb, 1, _TB)
    bias = (b1 + b2).reshape(1, H)
    Wqb = jnp.broadcast_to(Wq, (_B, H))

    tok = lambda i: jnp.minimum(i, nb - 1)
    voc = lambda i: jnp.maximum(i - nb, 0)

    z = pl.pallas_call(
        _make_body(nb),
        grid=(nb + nv,),
        in_specs=[
            pl.BlockSpec(memory_space=pltpu.SMEM),
            pl.BlockSpec((1, 1, _TB), lambda i: (tok(i), 0, 0)),
            pl.BlockSpec((_TB, H), lambda i: (tok(i), 0)),
            pl.BlockSpec((_B, H), lambda i: (0, 0)),
            pl.BlockSpec((H, H), lambda i: (0, 0)),
            pl.BlockSpec((H, H), lambda i: (0, 0)),
            pl.BlockSpec((_B, H), lambda i: (0, 0)),
            pl.BlockSpec((1, H), lambda i: (0, 0)),
            pl.BlockSpec((H, 2 * H), lambda i: (0, 0)),
            pl.BlockSpec((1, H), lambda i: (0, 0)),
            pl.BlockSpec((_VB, H), lambda i: (voc(i), 0)),
        ],
        out_specs=pl.BlockSpec((_B, _VB), lambda i: (0, voc(i))),
        out_shape=jax.ShapeDtypeStruct((_B, V), jnp.float32),
        scratch_shapes=[pltpu.VMEM((_B, H), jnp.float32),
                        pltpu.VMEM((_B, H), jnp.float32)],
    )(bq, batch3, session_embedding, v_n, W1, W2, Wqb, bias,
      W3, b3.reshape(1, H), item_weight)
    return z


# speedup vs baseline: 4.2519x; 1.1358x over previous
import jax
import jax.numpy as jnp
from jax import lax
from jax.experimental import pallas as pl
from jax.experimental.pallas import tpu as pltpu

_B = 16
_TB = 2048
_VB = 4096


def _make_body(nb):
    def _body(bq_ref, batch_ref, se_ref, vn_ref, W1_ref, W2_ref, Wqb_ref,
              bias_ref, W3_ref, b3_ref, iw_ref, z_ref, acc_ref, sh_ref):
        i = pl.program_id(0)

        @pl.when(i < nb)
        def _token():
            se = se_ref[...]
            tb = se.shape[0]
            brow = batch_ref[0]
            seg_iota = lax.broadcasted_iota(jnp.int32, (_B, tb), 0)
            ohT = (jnp.broadcast_to(brow, (_B, tb)) == seg_iota
                   ).astype(jnp.bfloat16)
            c = lax.dot_general(vn_ref[...], W1_ref[...],
                                (((1,), (1,)), ((), ())),
                                preferred_element_type=jnp.float32)
            c = (c + bias_ref[...]).astype(jnp.bfloat16)
            m = lax.dot_general(se, W2_ref[...], (((1,), (1,)), ((), ())),
                                preferred_element_type=jnp.float32)
            g = lax.dot_general(ohT, c, (((0,), (0,)), ((), ())),
                                preferred_element_type=jnp.float32)
            h = jax.nn.sigmoid(m + g).astype(jnp.bfloat16)
            alphaT = lax.dot_general(Wqb_ref[...], h, (((1,), (1,)), ((), ())),
                                     preferred_element_type=jnp.float32)
            wT = (ohT.astype(jnp.float32) * (alphaT + bq_ref[0])
                  ).astype(jnp.bfloat16)
            part = lax.dot_general(wT, se.astype(jnp.bfloat16),
                                   (((1,), (0,)), ((), ())),
                                   preferred_element_type=jnp.float32)

            @pl.when(i == 0)
            def _():
                acc_ref[...] = jnp.zeros_like(acc_ref)

            acc_ref[...] += part

            @pl.when(i == nb - 1)
            def _():
                cat = jnp.concatenate([vn_ref[...], acc_ref[...]], axis=1)
                sh_ref[...] = lax.dot_general(
                    cat, W3_ref[...], (((1,), (1,)), ((), ())),
                    preferred_element_type=jnp.float32) + b3_ref[...]

        @pl.when(i >= nb)
        def _vocab():
            z_ref[...] = lax.dot_general(sh_ref[...], iw_ref[...],
                                         (((1,), (1,)), ((), ())),
                                         preferred_element_type=jnp.float32)

    return _body


def kernel(session_embedding, batch, item_weight, W1, b1, W2, b2, Wq, bq, W3, b3):
    T, H = session_embedding.shape
    V = item_weight.shape[0]
    batch32 = batch.astype(jnp.int32)

    last = jnp.searchsorted(batch32, jnp.arange(_B, dtype=jnp.int32),
                            side="right").astype(jnp.int32) - 1
    v_n = jnp.take(session_embedding, last, axis=0, mode="clip")

    nb = T // _TB
    nv = pl.cdiv(V, _VB)
    batch3 = batch32.reshape(nb, 1, _TB)
    bias = (b1 + b2).reshape(1, H)
    Wqb = jnp.broadcast_to(Wq, (_B, H))

    tok = lambda i: jnp.minimum(i, nb - 1)
    voc = lambda i: jnp.maximum(i - nb, 0)

    z = pl.pallas_call(
        _make_body(nb),
        grid=(nb + nv,),
        in_specs=[
            pl.BlockSpec(memory_space=pltpu.SMEM),
            pl.BlockSpec((1, 1, _TB), lambda i: (tok(i), 0, 0)),
            pl.BlockSpec((_TB, H), lambda i: (tok(i), 0)),
            pl.BlockSpec((_B, H), lambda i: (0, 0)),
            pl.BlockSpec((H, H), lambda i: (0, 0)),
            pl.BlockSpec((H, H), lambda i: (0, 0)),
            pl.BlockSpec((_B, H), lambda i: (0, 0)),
            pl.BlockSpec((1, H), lambda i: (0, 0)),
            pl.BlockSpec((H, 2 * H), lambda i: (0, 0)),
            pl.BlockSpec((1, H), lambda i: (0, 0)),
            pl.BlockSpec((_VB, H), lambda i: (voc(i), 0)),
        ],
        out_specs=pl.BlockSpec((_B, _VB), lambda i: (0, voc(i))),
        out_shape=jax.ShapeDtypeStruct((_B, V), jnp.float32),
        scratch_shapes=[pltpu.VMEM((_B, H), jnp.float32),
                        pltpu.VMEM((_B, H), jnp.float32)],
    )(bq, batch3, session_embedding, v_n, W1, W2, Wqb, bias,
      W3, b3.reshape(1, H), item_weight)
    return z


# speedup vs baseline: 4.6373x; 1.0906x over previous
import jax
import jax.numpy as jnp
from jax import lax
from jax.experimental import pallas as pl
from jax.experimental.pallas import tpu as pltpu

_B = 16
_TB = 4096
_VB = 8192


def _make_body(nb):
    def _body(bq_ref, batch_ref, se_ref, vn_ref, W1_ref, W2_ref, Wqb_ref,
              bias_ref, W3_ref, b3_ref, iw_ref, z_ref, acc_ref, sh_ref):
        i = pl.program_id(0)

        @pl.when(i < nb)
        def _token():
            se = se_ref[...]
            tb = se.shape[0]
            brow = batch_ref[0]
            seg_iota = lax.broadcasted_iota(jnp.int32, (_B, tb), 0)
            ohT = (jnp.broadcast_to(brow, (_B, tb)) == seg_iota
                   ).astype(jnp.bfloat16)
            c = lax.dot_general(vn_ref[...], W1_ref[...],
                                (((1,), (1,)), ((), ())),
                                preferred_element_type=jnp.float32)
            c = (c + bias_ref[...]).astype(jnp.bfloat16)
            m = lax.dot_general(se, W2_ref[...], (((1,), (1,)), ((), ())),
                                preferred_element_type=jnp.float32)
            g = lax.dot_general(ohT, c, (((0,), (0,)), ((), ())),
                                preferred_element_type=jnp.float32)
            h = jax.nn.sigmoid(m + g).astype(jnp.bfloat16)
            alphaT = lax.dot_general(Wqb_ref[...], h, (((1,), (1,)), ((), ())),
                                     preferred_element_type=jnp.float32)
            wT = (ohT.astype(jnp.float32) * (alphaT + bq_ref[0])
                  ).astype(jnp.bfloat16)
            part = lax.dot_general(wT, se.astype(jnp.bfloat16),
                                   (((1,), (0,)), ((), ())),
                                   preferred_element_type=jnp.float32)

            @pl.when(i == 0)
            def _():
                acc_ref[...] = jnp.zeros_like(acc_ref)

            acc_ref[...] += part

            @pl.when(i == nb - 1)
            def _():
                cat = jnp.concatenate([vn_ref[...], acc_ref[...]], axis=1)
                sh_ref[...] = lax.dot_general(
                    cat, W3_ref[...], (((1,), (1,)), ((), ())),
                    preferred_element_type=jnp.float32) + b3_ref[...]

        @pl.when(i >= nb)
        def _vocab():
            z_ref[...] = lax.dot_general(sh_ref[...], iw_ref[...],
                                         (((1,), (1,)), ((), ())),
                                         preferred_element_type=jnp.float32)

    return _body


def kernel(session_embedding, batch, item_weight, W1, b1, W2, b2, Wq, bq, W3, b3):
    T, H = session_embedding.shape
    V = item_weight.shape[0]
    batch32 = batch.astype(jnp.int32)

    last = jnp.searchsorted(batch32, jnp.arange(_B, dtype=jnp.int32),
                            side="right").astype(jnp.int32) - 1
    v_n = jnp.take(session_embedding, last, axis=0, mode="clip")

    nb = T // _TB
    nv = pl.cdiv(V, _VB)
    batch3 = batch32.reshape(nb, 1, _TB)
    bias = (b1 + b2).reshape(1, H)
    Wqb = jnp.broadcast_to(Wq, (_B, H))

    tok = lambda i: jnp.minimum(i, nb - 1)
    voc = lambda i: jnp.maximum(i - nb, 0)

    z = pl.pallas_call(
        _make_body(nb),
        grid=(nb + nv,),
        in_specs=[
            pl.BlockSpec(memory_space=pltpu.SMEM),
            pl.BlockSpec((1, 1, _TB), lambda i: (tok(i), 0, 0)),
            pl.BlockSpec((_TB, H), lambda i: (tok(i), 0)),
            pl.BlockSpec((_B, H), lambda i: (0, 0)),
            pl.BlockSpec((H, H), lambda i: (0, 0)),
            pl.BlockSpec((H, H), lambda i: (0, 0)),
            pl.BlockSpec((_B, H), lambda i: (0, 0)),
            pl.BlockSpec((1, H), lambda i: (0, 0)),
            pl.BlockSpec((H, 2 * H), lambda i: (0, 0)),
            pl.BlockSpec((1, H), lambda i: (0, 0)),
            pl.BlockSpec((_VB, H), lambda i: (voc(i), 0)),
        ],
        out_specs=pl.BlockSpec((_B, _VB), lambda i: (0, voc(i))),
        out_shape=jax.ShapeDtypeStruct((_B, V), jnp.float32),
        scratch_shapes=[pltpu.VMEM((_B, H), jnp.float32),
                        pltpu.VMEM((_B, H), jnp.float32)],
    )(bq, batch3, session_embedding, v_n, W1, W2, Wqb, bias,
      W3, b3.reshape(1, H), item_weight)
    return z
